# Initial kernel scaffold; baseline (speedup 1.0000x reference)
import jax
import jax.numpy as jnp
from jax.experimental import pallas as pl


def kernel(mask, embed_weight):
    raise NotImplementedError("write your pallas kernel here")


# speedup vs baseline: 1.3666x; 1.3666x over previous
import jax
import jax.numpy as jnp
from jax.experimental import pallas as pl


def _bcast_body(emb_ref, out_ref):
    out_ref[...] = emb_ref[...][None]


def kernel(mask, embed_weight):
    bs, t = mask.shape
    n_embed, d = embed_weight.shape

    out = pl.pallas_call(
        _bcast_body,
        grid=(bs,),
        in_specs=[pl.BlockSpec((t, d), lambda b: (0, 0))],
        out_specs=pl.BlockSpec((1, t, d), lambda b: (b, 0, 0)),
        out_shape=jax.ShapeDtypeStruct((bs, t, d), embed_weight.dtype),
    )(embed_weight[:t])
    return out


# speedup vs baseline: 1.5538x; 1.1370x over previous
import jax
import jax.numpy as jnp
from jax.experimental import pallas as pl
from jax.experimental.pallas import tpu as pltpu


def _make_body(bs):
    def body(emb_ref, out_ref, sem):
        copies = [
            pltpu.make_async_copy(emb_ref, out_ref.at[b], sem)
            for b in range(bs)
        ]
        for c in copies:
            c.start()
        for c in copies:
            c.wait()

    return body


def kernel(mask, embed_weight):
    bs, t = mask.shape
    n_embed, d = embed_weight.shape

    out = pl.pallas_call(
        _make_body(bs),
        in_specs=[pl.BlockSpec(memory_space=pltpu.MemorySpace.VMEM)],
        out_specs=pl.BlockSpec(memory_space=pl.ANY),
        out_shape=jax.ShapeDtypeStruct((bs, t, d), embed_weight.dtype),
        scratch_shapes=[pltpu.SemaphoreType.DMA],
    )(embed_weight[:t])
    return out
